# baseline (speedup 1.0000x reference)
import functools

import jax
import jax.numpy as jnp
from jax import lax
from jax.experimental import pallas as pl
from jax.experimental.pallas import tpu as pltpu
from jax.experimental.pallas import tpu_sc as plsc

_B, _C, _H, _W = 8, 192, 224, 224
_ROWS = _B * _C
_D = _H * _W
_NC = 2
_RPC = _ROWS // _NC
_NBUF = 16

_mesh = plsc.ScalarSubcoreMesh(axis_name="c", num_cores=_NC)


@functools.partial(
    pl.kernel,
    mesh=_mesh,
    out_type=jax.ShapeDtypeStruct((_ROWS, _D), jnp.float32),
    scratch_types=[pltpu.SemaphoreType.DMA((_NBUF,))],
)
def _reverse_rows(in_hbm, out_hbm, sems):
    cid = lax.axis_index("c")
    row0 = cid * _RPC

    def src_of(r):
        b = r // _C
        c = lax.rem(r, _C)
        return b * _C + (_C - 1 - c)

    def start(i):
        slot = lax.rem(i, _NBUF)
        r = row0 + i
        pltpu.async_copy(in_hbm.at[src_of(r)], out_hbm.at[r], sems.at[slot])

    def wait(i):
        slot = lax.rem(i, _NBUF)
        r = row0 + i
        pltpu.make_async_copy(in_hbm.at[src_of(r)], out_hbm.at[r],
                              sems.at[slot]).wait()

    for j in range(_NBUF):
        start(j)

    def body(i, carry):
        wait(i)
        pl.when(i + _NBUF < _RPC)(lambda: start(i + _NBUF))
        return carry

    lax.fori_loop(0, _RPC - _NBUF, body, 0)
    for j in range(_RPC - _NBUF, _RPC):
        wait(j)


def kernel(input):
    x = input.reshape(_ROWS, _D)
    y = _reverse_rows(x)
    return y.reshape(_B, _C, _H, _W)


# speedup vs baseline: 11.6170x; 11.6170x over previous
import functools

import jax
import jax.numpy as jnp
from jax import lax
from jax.experimental import pallas as pl
from jax.experimental.pallas import tpu as pltpu
from jax.experimental.pallas import tpu_sc as plsc

_B, _C, _H, _W = 8, 192, 224, 224
_ROWS = _B * _C
_D = _H * _W
_NC, _NS = 2, 16
_NW = _NC * _NS
_RPW = _ROWS // _NW

_mesh = plsc.VectorSubcoreMesh(core_axis_name="c", subcore_axis_name="s")

_SPLIT = 2
_CH = _D // _SPLIT
_NBUF = 4
_T = _RPW * _SPLIT


@functools.partial(
    pl.kernel,
    mesh=_mesh,
    out_type=jax.ShapeDtypeStruct((_ROWS, _D), jnp.float32),
    scratch_types=[
        pltpu.VMEM_SHARED((_NS, _NBUF, _CH), jnp.float32),
        pltpu.SemaphoreType.DMA((_NBUF,)),
        pltpu.SemaphoreType.DMA((_NBUF,)),
    ],
)
def _reverse_rows(in_hbm, out_hbm, shared, in_sems, out_sems):
    cid = lax.axis_index("c")
    sid = lax.axis_index("s")
    wid = sid * _NC + cid
    base = wid * _RPW
    bufs = shared.at[sid]

    def src_slice(i):
        r = base + i // _SPLIT
        k = lax.rem(i, _SPLIT)
        b = r // _C
        c = lax.rem(r, _C)
        src = b * _C + (_C - 1 - c)
        return in_hbm.at[src, pl.ds(k * _CH, _CH)]

    def dst_slice(i):
        r = base + i // _SPLIT
        k = lax.rem(i, _SPLIT)
        return out_hbm.at[r, pl.ds(k * _CH, _CH)]

    def start_in(i):
        slot = lax.rem(i, _NBUF)
        pltpu.async_copy(src_slice(i), bufs.at[slot], in_sems.at[slot])

    def wait_in(i):
        slot = lax.rem(i, _NBUF)
        pltpu.make_async_copy(src_slice(i), bufs.at[slot],
                              in_sems.at[slot]).wait()

    def start_out(i):
        slot = lax.rem(i, _NBUF)
        pltpu.async_copy(bufs.at[slot], dst_slice(i), out_sems.at[slot])

    def wait_out(i):
        slot = lax.rem(i, _NBUF)
        pltpu.make_async_copy(bufs.at[slot], dst_slice(i),
                              out_sems.at[slot]).wait()

    for j in range(_NBUF - 1):
        start_in(j)

    def body(i, carry):
        wait_in(i)
        start_out(i)
        pl.when(jnp.logical_and(i >= 1, i + _NBUF - 1 < _T))(
            lambda: wait_out(i - 1))
        pl.when(i + _NBUF - 1 < _T)(lambda: start_in(i + _NBUF - 1))
        return carry

    lax.fori_loop(0, _T, body, 0)
    for j in range(_T - _NBUF, _T):
        wait_out(j)


def kernel(input):
    x = input.reshape(_ROWS, _D)
    y = _reverse_rows(x)
    return y.reshape(_B, _C, _H, _W)


# speedup vs baseline: 11.6981x; 1.0070x over previous
import functools

import jax
import jax.numpy as jnp
from jax import lax
from jax.experimental import pallas as pl
from jax.experimental.pallas import tpu as pltpu
from jax.experimental.pallas import tpu_sc as plsc

_B, _C, _H, _W = 8, 192, 224, 224
_ROWS = _B * _C
_D = _H * _W
_NC, _NS = 2, 16
_NW = _NC * _NS
_RPW = _ROWS // _NW

_mesh = plsc.VectorSubcoreMesh(core_axis_name="c", subcore_axis_name="s")

_SPLIT = 4
_CH = _D // _SPLIT
_NBUF = 8
_T = _RPW * _SPLIT


@functools.partial(
    pl.kernel,
    mesh=_mesh,
    out_type=jax.ShapeDtypeStruct((_ROWS, _D), jnp.float32),
    scratch_types=[
        pltpu.VMEM_SHARED((_NS, _NBUF, _CH), jnp.float32),
        pltpu.SemaphoreType.DMA((_NBUF,)),
        pltpu.SemaphoreType.DMA((_NBUF,)),
    ],
)
def _reverse_rows(in_hbm, out_hbm, shared, in_sems, out_sems):
    cid = lax.axis_index("c")
    sid = lax.axis_index("s")
    wid = sid * _NC + cid
    base = wid * _RPW
    bufs = shared.at[sid]

    def src_slice(i):
        r = base + i // _SPLIT
        k = lax.rem(i, _SPLIT)
        b = r // _C
        c = lax.rem(r, _C)
        src = b * _C + (_C - 1 - c)
        return in_hbm.at[src, pl.ds(k * _CH, _CH)]

    def dst_slice(i):
        r = base + i // _SPLIT
        k = lax.rem(i, _SPLIT)
        return out_hbm.at[r, pl.ds(k * _CH, _CH)]

    def start_in(i):
        slot = lax.rem(i, _NBUF)
        pltpu.async_copy(src_slice(i), bufs.at[slot], in_sems.at[slot])

    def wait_in(i):
        slot = lax.rem(i, _NBUF)
        pltpu.make_async_copy(src_slice(i), bufs.at[slot],
                              in_sems.at[slot]).wait()

    def start_out(i):
        slot = lax.rem(i, _NBUF)
        pltpu.async_copy(bufs.at[slot], dst_slice(i), out_sems.at[slot])

    def wait_out(i):
        slot = lax.rem(i, _NBUF)
        pltpu.make_async_copy(bufs.at[slot], dst_slice(i),
                              out_sems.at[slot]).wait()

    for j in range(_NBUF - 1):
        start_in(j)

    def body(i, carry):
        wait_in(i)
        start_out(i)
        pl.when(jnp.logical_and(i >= 1, i + _NBUF - 1 < _T))(
            lambda: wait_out(i - 1))
        pl.when(i + _NBUF - 1 < _T)(lambda: start_in(i + _NBUF - 1))
        return carry

    lax.fori_loop(0, _T, body, 0)
    for j in range(_T - _NBUF, _T):
        wait_out(j)


def kernel(input):
    x = input.reshape(_ROWS, _D)
    y = _reverse_rows(x)
    return y.reshape(_B, _C, _H, _W)
